# baseline (speedup 1.0000x reference)
import functools

import jax
import jax.numpy as jnp
import numpy as np
from jax import lax
from jax.experimental import pallas as pl
from jax.experimental.pallas import tpu as pltpu
from jax.experimental.pallas import tpu_sc as plsc

F32 = jnp.float32
I32 = jnp.int32

_N = 10000
_D = 128
_DE = 16
_M = 320000

_NC = 2
_NS = 16
_NW = _NC * _NS

_NP = 10240
_NPW = _NP // _NW
_EPW = 10240
_MP = _NW * _EPW
_MPAD = _MP + 16384
_CHA = 32
_NBUF = 4
_E0 = 12416
_E1 = 2 * 10240 - _E0
_CHB = 128
_NCHA = _EPW // _CHA
_NCHMAX = max(_E0, _E1) // _CHA
_E0B = 11008
_E1B = 2 * 10240 - _E0B
_NCHB = _EPW // _CHB
_NCHBMAX = max(_E0B, _E1B) // _CHB

_NEG = float("-inf")


def _upd_pair(kref, aref, t16, k16, a16):
    m = k16 > plsc.load_gather(kref, [t16])
    for _ in range(3):
        plsc.store_scatter(kref, [t16], k16, mask=m)
        plsc.store_scatter(aref, [t16], a16, mask=m)
        gk = plsc.load_gather(kref, [t16])
        ga = plsc.load_gather(aref, [t16])
        again = jnp.logical_or(
            k16 > gk, jnp.logical_and(k16 == gk, ga != a16))
        m = jnp.logical_and(again, m)


def _upd_val(kref, t16, k16):
    m = k16 > plsc.load_gather(kref, [t16])
    for _ in range(3):
        plsc.store_scatter(kref, [t16], k16, mask=m)
        m = jnp.logical_and(k16 > plsc.load_gather(kref, [t16]), m)


def _worker_id():
    return lax.axis_index("s") * _NC + lax.axis_index("c")


def _butterfly16(vecs):
    lanes = lax.iota(I32, 16)
    for mask in (8, 4, 2, 1):
        perm = jnp.bitwise_xor(lanes, mask)
        sel = (lanes & mask) == 0
        half = len(vecs) // 2
        nxt = []
        for i in range(half):
            a = vecs[i]
            b = vecs[i + half]
            ra = a + jnp.take_along_axis(a, perm, axis=0)
            rb = b + jnp.take_along_axis(b, perm, axis=0)
            nxt.append(jnp.where(sel, ra, rb))
        vecs = nxt
    return vecs[0]


def _prep_body(x1_ref, x2_ref, e1_ref, e2_ref, o1_ref, o2_ref, o3_ref, o4_ref):
    for x_ref, o_ref in ((x1_ref, o1_ref), (x2_ref, o2_ref),
                         (e1_ref, o3_ref), (e2_ref, o4_ref)):
        x = x_ref[...]
        n = jnp.sqrt(jnp.sum(x * x, axis=2))
        o_ref[...] = 1.0 / jnp.maximum(n, 1e-8)


def _inv_norms(x1p, x2p, e1p, e2p):
    g = _NP // 1024
    outs = pl.pallas_call(
        _prep_body,
        grid=(g,),
        in_specs=[
            pl.BlockSpec((8, 128, _D), lambda i: (i, 0, 0)),
            pl.BlockSpec((8, 128, _D), lambda i: (i, 0, 0)),
            pl.BlockSpec((8, 128, _DE), lambda i: (i, 0, 0)),
            pl.BlockSpec((8, 128, _DE), lambda i: (i, 0, 0)),
        ],
        out_specs=[pl.BlockSpec((8, 128), lambda i: (i, 0))] * 4,
        out_shape=[jax.ShapeDtypeStruct((8 * g, 128), F32)] * 4,
    )(x1p.reshape(8 * g, 128, _D), x2p.reshape(8 * g, 128, _D),
      e1p.reshape(8 * g, 128, _DE), e2p.reshape(8 * g, 128, _DE))
    return tuple(o.reshape(_NP) for o in outs)


def _sc_pass_a(x1, x2, srgood, engood, inv1, inv2):
    mesh = plsc.VectorSubcoreMesh(core_axis_name="c", subcore_axis_name="s")

    @functools.partial(
        pl.kernel,
        out_type=(
            jax.ShapeDtypeStruct((_NW, _NP), F32),
            jax.ShapeDtypeStruct((_NW, _NP), I32),
            jax.ShapeDtypeStruct((_NW, _NP), F32),
            jax.ShapeDtypeStruct((_NW, _NP), I32),
        ),
        mesh=mesh,
        compiler_params=pltpu.CompilerParams(needs_layout_passes=False, use_tc_tiling_on_sc=False),
        scratch_types=[
            pltpu.VMEM((_NCHMAX, _CHA), I32),
            pltpu.VMEM((_NCHMAX, _CHA), I32),
            pltpu.VMEM((_NP,), F32),
            pltpu.VMEM((_NP,), F32),
            pltpu.VMEM((_NP,), F32),
            pltpu.VMEM((_NP,), I32),
            pltpu.VMEM((_NP,), F32),
            pltpu.VMEM((_NP,), I32),
            pltpu.VMEM((_NBUF * _CHA, _D), F32),
            pltpu.VMEM((_NBUF * _CHA, _D), F32),
            pltpu.SemaphoreType.DMA,
            pltpu.SemaphoreType.DMA,
        ],
    )
    def k(x1h, x2h, srch, endh, inv1h, inv2h,
          ks1o, js1o, ks2o, is2o,
          sidx, eidx, inv1v, inv2v, bk1, ba1, bk2, ba2, x1r, x2r, sem1, sem2):
        w = _worker_id()
        cid = lax.axis_index("c")
        sid = lax.axis_index("s")
        ebase = sid * (2 * _EPW) + cid * _E0
        ecnt = jnp.where(cid == 0, _E0, _E1)
        nch = ecnt // _CHA
        chbase = ebase // _CHA
        pltpu.sync_copy(srch.at[pl.ds(chbase, _NCHMAX)], sidx)
        pltpu.sync_copy(endh.at[pl.ds(chbase, _NCHMAX)], eidx)
        pltpu.sync_copy(inv1h, inv1v)
        pltpu.sync_copy(inv2h, inv2v)

        neg = jnp.full((16,), _NEG, F32)
        zero = jnp.zeros((16,), I32)

        def init(i, _):
            bk1[pl.ds(i * 16, 16)] = neg
            bk2[pl.ds(i * 16, 16)] = neg
            ba1[pl.ds(i * 16, 16)] = zero
            ba2[pl.ds(i * 16, 16)] = zero
            return 0

        lax.fori_loop(0, _NP // 16, init, 0)

        def issue(c, b):
            pltpu.async_copy(x1h.at[sidx.at[c]],
                             x1r.at[pl.ds(b * _CHA, _CHA)], sem1)
            pltpu.async_copy(x2h.at[eidx.at[c]],
                             x2r.at[pl.ds(b * _CHA, _CHA)], sem2)

        def wait(b):
            pltpu.make_async_copy(x1h.at[sidx.at[0]],
                                  x1r.at[pl.ds(b * _CHA, _CHA)], sem1).wait()
            pltpu.make_async_copy(x2h.at[eidx.at[0]],
                                  x2r.at[pl.ds(b * _CHA, _CHA)], sem2).wait()

        def compute(c, b):
            rowoff = b * _CHA

            def grp(g, _):
                s16 = sidx[c, pl.ds(g * 16, 16)]
                e16 = eidx[c, pl.ds(g * 16, 16)]
                parts = [
                    x1r[rowoff + g * 16 + e, pl.ds(0, 16)]
                    * x2r[rowoff + g * 16 + e, pl.ds(0, 16)]
                    for e in range(16)
                ]
                for k in range(1, _D // 16):
                    for e in range(16):
                        row = rowoff + g * 16 + e
                        parts[e] = parts[e] + (x1r[row, pl.ds(k * 16, 16)]
                                               * x2r[row, pl.ds(k * 16, 16)])
                acc = _butterfly16(parts)
                i2 = plsc.load_gather(inv2v, [e16])
                i1 = plsc.load_gather(inv1v, [s16])
                _upd_pair(bk1, ba1, s16, acc * i2, e16)
                _upd_pair(bk2, ba2, e16, acc * i1, s16)
                return 0

            lax.fori_loop(0, _CHA // 16, grp, 0)

        for b in range(_NBUF):
            issue(b, b)

        def chunkn(cc, _):
            for b in range(_NBUF):
                c = cc * _NBUF + b
                wait(b)
                compute(c, b)

                @pl.when(c + _NBUF < nch)
                def _():
                    issue(c + _NBUF, b)

            return 0

        lax.fori_loop(0, nch // _NBUF, chunkn, 0)

        pltpu.sync_copy(bk1, ks1o.at[w])
        pltpu.sync_copy(ba1, js1o.at[w])
        pltpu.sync_copy(bk2, ks2o.at[w])
        pltpu.sync_copy(ba2, is2o.at[w])

    return k(x1, x2, srgood, engood, inv1, inv2)


def _sc_pass_b(ea1, ea2, srood, engood, ive1, ive2):
    mesh = plsc.VectorSubcoreMesh(core_axis_name="c", subcore_axis_name="s")

    @functools.partial(
        pl.kernel,
        out_type=(
            jax.ShapeDtypeStruct((_NW, _NP), F32),
            jax.ShapeDtypeStruct((_NW, _NP), I32),
            jax.ShapeDtypeStruct((_NW, _NP), F32),
            jax.ShapeDtypeStruct((_NW, _NP), I32),
            jax.ShapeDtypeStruct((_NW, _NP), F32),
            jax.ShapeDtypeStruct((_NW, _NP), F32),
        ),
        mesh=mesh,
        compiler_params=pltpu.CompilerParams(needs_layout_passes=False, use_tc_tiling_on_sc=False),
        scratch_types=[
            pltpu.VMEM((_NCHBMAX, _CHB), I32),
            pltpu.VMEM((_NCHBMAX, _CHB), I32),
            pltpu.VMEM((_NP,), F32),
            pltpu.VMEM((_NP,), F32),
            pltpu.VMEM((_NP,), F32),
            pltpu.VMEM((_NP,), I32),
            pltpu.VMEM((_NP,), F32),
            pltpu.VMEM((_NP,), I32),
            pltpu.VMEM((_NP,), F32),
            pltpu.VMEM((_NP,), F32),
            pltpu.VMEM((2 * _CHB, _DE), F32),
            pltpu.VMEM((2 * _CHB, _DE), F32),
            pltpu.SemaphoreType.DMA,
            pltpu.SemaphoreType.DMA,
        ],
    )
    def k(e1h, e2h, srch, endh, ive1h, ive2h,
          kd1o, jd1o, kd2o, id2o, kc1o, kc2o,
          sidx, eidx, ie1v, ie2v, bkd1, bad1, bkd2, bad2, bkc1, bkc2,
          e1r, e2r, sem1, sem2):
        w = _worker_id()
        cid = lax.axis_index("c")
        sid = lax.axis_index("s")
        ebase = sid * (2 * _EPW) + cid * _E0B
        ecnt = jnp.where(cid == 0, _E0B, _E1B)
        nch = ecnt // _CHB
        chbase = ebase // _CHB
        pltpu.sync_copy(srch.at[pl.ds(chbase, _NCHBMAX)], sidx)
        pltpu.sync_copy(endh.at[pl.ds(chbase, _NCHBMAX)], eidx)
        pltpu.sync_copy(ive1h, ie1v)
        pltpu.sync_copy(ive2h, ie2v)

        neg = jnp.full((16,), _NEG, F32)
        zero = jnp.zeros((16,), I32)

        def init(i, _):
            bkd1[pl.ds(i * 16, 16)] = neg
            bkd2[pl.ds(i * 16, 16)] = neg
            bkc1[pl.ds(i * 16, 16)] = neg
            bkc2[pl.ds(i * 16, 16)] = neg
            bad1[pl.ds(i * 16, 16)] = zero
            bad2[pl.ds(i * 16, 16)] = zero
            return 0

        lax.fori_loop(0, _NP // 16, init, 0)

        def issue(c, b):
            pltpu.async_copy(e1h.at[sidx.at[c]],
                             e1r.at[pl.ds(b * _CHB, _CHB)], sem1)
            pltpu.async_copy(e2h.at[eidx.at[c]],
                             e2r.at[pl.ds(b * _CHB, _CHB)], sem2)

        def wait(b):
            pltpu.make_async_copy(e1h.at[sidx.at[0]],
                                  e1r.at[pl.ds(b * _CHB, _CHB)], sem1).wait()
            pltpu.make_async_copy(e2h.at[eidx.at[0]],
                                  e2r.at[pl.ds(b * _CHB, _CHB)], sem2).wait()

        def compute(c, b):
            rowoff = b * _CHB

            def grp(g, _):
                s16 = sidx[c, pl.ds(g * 16, 16)]
                e16 = eidx[c, pl.ds(g * 16, 16)]
                sparts = []
                tparts = []
                for e in range(16):
                    row = rowoff + g * 16 + e
                    a = e1r[row, :]
                    b_ = e2r[row, :]
                    df = a - b_ + 1e-6
                    sparts.append(df * df)
                    tparts.append(a * b_)
                sacc = _butterfly16(sparts)
                tacc = _butterfly16(tparts)
                kd = -sacc
                i1 = plsc.load_gather(ie1v, [s16])
                i2 = plsc.load_gather(ie2v, [e16])
                kc = tacc * i1 * i2
                _upd_pair(bkd1, bad1, s16, kd, e16)
                _upd_pair(bkd2, bad2, e16, kd, s16)
                _upd_val(bkc1, s16, kc)
                _upd_val(bkc2, e16, kc)
                return 0

            lax.fori_loop(0, _CHB // 16, grp, 0)

        issue(0, 0)
        issue(1, 1)

        def chunk2(cc, _):
            c0 = 2 * cc
            wait(0)
            compute(c0, 0)

            @pl.when(c0 + 2 < nch)
            def _():
                issue(c0 + 2, 0)

            wait(1)
            compute(c0 + 1, 1)

            @pl.when(c0 + 3 < nch)
            def _():
                issue(c0 + 3, 1)

            return 0

        lax.fori_loop(0, nch // 2, chunk2, 0)

        pltpu.sync_copy(bkd1, kd1o.at[w])
        pltpu.sync_copy(bad1, jd1o.at[w])
        pltpu.sync_copy(bkd2, kd2o.at[w])
        pltpu.sync_copy(bad2, id2o.at[w])
        pltpu.sync_copy(bkc1, kc1o.at[w])
        pltpu.sync_copy(bkc2, kc2o.at[w])

    return k(ea1, ea2, srood, engood, ive1, ive2)


def _sc_pass_c(ks1, js1, ks2, is2, kd1, jd1, kd2, id2, kc1, kc2,
               x1, x2, ea1, ea2, fb):
    mesh = plsc.VectorSubcoreMesh(core_axis_name="c", subcore_axis_name="s")

    @functools.partial(
        pl.kernel,
        out_type=(
            jax.ShapeDtypeStruct((_NP, _D), F32),
            jax.ShapeDtypeStruct((_NP, _D), F32),
            jax.ShapeDtypeStruct((_NP, _DE), F32),
            jax.ShapeDtypeStruct((_NP, _DE), F32),
            jax.ShapeDtypeStruct((8, _NP), F32),
        ),
        mesh=mesh,
        compiler_params=pltpu.CompilerParams(needs_layout_passes=False, use_tc_tiling_on_sc=False),
        scratch_types=[
            pltpu.VMEM((_NW, _NPW), F32),
            pltpu.VMEM((_NW, _NPW), I32),
            pltpu.VMEM((_NPW,), F32),
            pltpu.VMEM((_NPW // 64, 64), I32),
            pltpu.VMEM((16,), I32),
            pltpu.VMEM((64, _D), F32),
            pltpu.VMEM((64, _DE), F32),
            pltpu.SemaphoreType.DMA,
        ],
    )
    def k(ks1h, js1h, ks2h, is2h, kd1h, jd1h, kd2h, id2h, kc1h, kc2h,
          x1h, x2h, e1h, e2h, fbh,
          x2go, x1go, e2go, e1go, valso,
          kslab, aslab, mk, ja, fbv, gbx, gbe, sem):
        w = _worker_id()
        nbase = w * _NPW
        pltpu.sync_copy(fbh, fbv)
        fb_all = fbv[pl.ds(0, 16)]

        def merge(khbm, ahbm, fb_lane):
            pltpu.sync_copy(khbm.at[:, pl.ds(nbase, _NPW)], kslab)
            if ahbm is not None:
                pltpu.sync_copy(ahbm.at[:, pl.ds(nbase, _NPW)], aslab)
            fbs = fb_all[fb_lane]
            for v in range(_NPW // 16):
                bk = kslab[0, pl.ds(v * 16, 16)]
                ba = aslab[0, pl.ds(v * 16, 16)]

                def red(kw, st):
                    bk_, ba_ = st
                    kk = kslab[kw, pl.ds(v * 16, 16)]
                    aa = aslab[kw, pl.ds(v * 16, 16)]
                    m = kk > bk_
                    return (jnp.where(m, kk, bk_), jnp.where(m, aa, ba_))

                bk, ba = lax.fori_loop(1, _NW, red, (bk, ba))
                emp = bk == _NEG
                ba = jnp.where(emp, jnp.full((16,), fbs, I32), ba)
                mk[pl.ds(v * 16, 16)] = bk
                ja[v // 4, pl.ds((v % 4) * 16, 16)] = ba

        def gather_rows(table, gbuf, out):
            for i in range(_NPW // 64):
                pltpu.async_copy(table.at[ja.at[i]], gbuf, sem).wait()
                pltpu.sync_copy(gbuf, out.at[pl.ds(nbase + i * 64, 64)])

        def save_vals(r):
            pltpu.sync_copy(mk, valso.at[r, pl.ds(nbase, _NPW)])

        merge(ks1h, js1h, 0)
        gather_rows(x2h, gbx, x2go)
        merge(ks2h, is2h, 1)
        gather_rows(x1h, gbx, x1go)
        merge(kd1h, jd1h, 0)
        gather_rows(e2h, gbe, e2go)
        save_vals(0)
        merge(kd2h, id2h, 1)
        gather_rows(e1h, gbe, e1go)
        save_vals(1)
        merge(kc1h, None, 0)
        save_vals(2)
        merge(kc2h, None, 1)
        save_vals(3)

    return k(ks1, js1, ks2, is2, kd1, jd1, kd2, id2, kc1, kc2,
             x1, x2, ea1, ea2, fb)


def _final_body(x1_ref, x2_ref, e1_ref, e2_ref, x2g_ref, x1g_ref,
                e2g_ref, e1g_ref, vals_ref, we_ref, be_ref, wn_ref, bn_ref,
                oe1_ref, oe2_ref, ox1_ref, ox2_ref):
    x1 = x1_ref[...]
    x2 = x2_ref[...]
    e1 = e1_ref[...]
    e2 = e2_ref[...]
    x2g = x2g_ref[...]
    x1g = x1g_ref[...]
    e2g = e2g_ref[...]
    e1g = e1g_ref[...]
    we = we_ref[...]
    wn = wn_ref[...]
    be = be_ref[0]
    bn = bn_ref[0]

    def dot(a, b):
        return lax.dot_general(a, b, (((1,), (0,)), ((), ())),
                               preferred_element_type=F32)

    oe1_ref[...] = (dot(e1, we[0:_DE]) + dot(x1, we[_DE:_DE + _D])
                    + dot(x2g, we[_DE + _D:_DE + 2 * _D]) + be)
    oe2_ref[...] = (dot(e2, we[0:_DE]) + dot(x2, we[_DE:_DE + _D])
                    + dot(x1g, we[_DE + _D:_DE + 2 * _D]) + be)

    kd1 = vals_ref[0]
    kd2 = vals_ref[1]
    kc1 = vals_ref[2]
    kc2 = vals_ref[3]
    dis1 = jnp.where(jnp.isfinite(kd1), -jnp.sqrt(jnp.maximum(-kd1, 0.0)), 0.0)
    dis2 = jnp.where(jnp.isfinite(kd2), -jnp.sqrt(jnp.maximum(-kd2, 0.0)), 0.0)
    sim1 = jnp.where(jnp.isfinite(kc1), kc1, 0.0)
    sim2 = jnp.where(jnp.isfinite(kc2), kc2, 0.0)

    ox1_ref[...] = (dot(x1, wn[0:_D]) + dot(e1, wn[_D:_D + _DE])
                    + dot(e2g, wn[_D + _DE:_D + 2 * _DE])
                    + dis1[:, None] * wn[_D + 2 * _DE]
                    + sim1[:, None] * wn[_D + 2 * _DE + 1] + bn)
    ox2_ref[...] = (dot(x2, wn[0:_D]) + dot(e2, wn[_D:_D + _DE])
                    + dot(e1g, wn[_D + _DE:_D + 2 * _DE])
                    + dis2[:, None] * wn[_D + 2 * _DE]
                    + sim2[:, None] * wn[_D + 2 * _DE + 1] + bn)


def _final(x1p, x2p, e1p, e2p, x2g, x1g, e2g, e1g, vals,
           W_edge, b_edge, W_node, b_node):
    blk = 1024
    g = _NP // blk
    kd = 168
    return pl.pallas_call(
        _final_body,
        grid=(g,),
        in_specs=[
            pl.BlockSpec((blk, _D), lambda i: (i, 0)),
            pl.BlockSpec((blk, _D), lambda i: (i, 0)),
            pl.BlockSpec((blk, _DE), lambda i: (i, 0)),
            pl.BlockSpec((blk, _DE), lambda i: (i, 0)),
            pl.BlockSpec((blk, _D), lambda i: (i, 0)),
            pl.BlockSpec((blk, _D), lambda i: (i, 0)),
            pl.BlockSpec((blk, _DE), lambda i: (i, 0)),
            pl.BlockSpec((blk, _DE), lambda i: (i, 0)),
            pl.BlockSpec((8, blk), lambda i: (0, i)),
            pl.BlockSpec((_DE + 2 * _D, _DE), lambda i: (0, 0)),
            pl.BlockSpec((1, _DE), lambda i: (0, 0)),
            pl.BlockSpec((kd, _D), lambda i: (0, 0)),
            pl.BlockSpec((1, _D), lambda i: (0, 0)),
        ],
        out_specs=[
            pl.BlockSpec((blk, _DE), lambda i: (i, 0)),
            pl.BlockSpec((blk, _DE), lambda i: (i, 0)),
            pl.BlockSpec((blk, _D), lambda i: (i, 0)),
            pl.BlockSpec((blk, _D), lambda i: (i, 0)),
        ],
        out_shape=[
            jax.ShapeDtypeStruct((_NP, _DE), F32),
            jax.ShapeDtypeStruct((_NP, _DE), F32),
            jax.ShapeDtypeStruct((_NP, _D), F32),
            jax.ShapeDtypeStruct((_NP, _D), F32),
        ],
    )(x1p, x2p, e1p, e2p, x2g, x1g, e2g, e1g, vals,
      W_edge, b_edge.reshape(1, _DE), W_node, b_node.reshape(1, _D))


def kernel(x1, x2, edge_attr1, edge_attr2, matching_idx, W_edge, b_edge,
           W_node, b_node):
    src = matching_idx[0].astype(I32)
    end = matching_idx[1].astype(I32)
    pad = _MPAD - _M
    srcp = jnp.concatenate([src, jnp.broadcast_to(src[-1], (pad,))])
    endp = jnp.concatenate([end, jnp.broadcast_to(end[-1], (pad,))])

    npad = _NP - _N
    x1p = jnp.pad(x1, ((0, npad), (0, 0)))
    x2p = jnp.pad(x2, ((0, npad), (0, 0)))
    e1p = jnp.pad(edge_attr1, ((0, npad), (0, 0)))
    e2p = jnp.pad(edge_attr2, ((0, npad), (0, 0)))

    inv1, inv2, ive1, ive2 = _inv_norms(x1p, x2p, e1p, e2p)

    ks1, js1, ks2, is2 = _sc_pass_a(
        x1, x2, srcp.reshape(-1, _CHA), endp.reshape(-1, _CHA), inv1, inv2)
    kd1, jd1, kd2, id2, kc1, kc2 = _sc_pass_b(
        edge_attr1, edge_attr2, srcp.reshape(-1, _CHB),
        endp.reshape(-1, _CHB), ive1, ive2)

    fb = jnp.concatenate([end[-1:], src[-1:], jnp.zeros((14,), I32)])
    x2g, x1g, e2g, e1g, vals = _sc_pass_c(
        ks1, js1, ks2, is2, kd1, jd1, kd2, id2, kc1, kc2,
        x1, x2, edge_attr1, edge_attr2, fb)

    Wn_pad = jnp.pad(W_node, ((0, 168 - (_D + 2 * _DE + 2)), (0, 0)))
    oe1, oe2, ox1, ox2 = _final(
        x1p, x2p, e1p, e2p, x2g, x1g, e2g, e1g, vals,
        W_edge, b_edge, Wn_pad, b_node)

    return (ox1[:_N], ox2[:_N], oe1[:_N], oe2[:_N])


# speedup vs baseline: 1.0071x; 1.0071x over previous
import functools

import jax
import jax.numpy as jnp
import numpy as np
from jax import lax
from jax.experimental import pallas as pl
from jax.experimental.pallas import tpu as pltpu
from jax.experimental.pallas import tpu_sc as plsc

F32 = jnp.float32
I32 = jnp.int32

_N = 10000
_D = 128
_DE = 16
_M = 320000

_NC = 2
_NS = 16
_NW = _NC * _NS

_NP = 10240
_NPW = _NP // _NW
_EPW = 10240
_MP = _NW * _EPW
_MPAD = _MP + 16384
_CHA = 32
_NBUF = 4
_E0 = 12032
_E1 = 2 * 10240 - _E0
_CHB = 128
_NCHA = _EPW // _CHA
_NCHMAX = max(_E0, _E1) // _CHA
_NCHB = _EPW // _CHB

_NEG = float("-inf")


def _upd_pair(kref, aref, t16, k16, a16):
    m = k16 > plsc.load_gather(kref, [t16])
    for _ in range(3):
        plsc.store_scatter(kref, [t16], k16, mask=m)
        plsc.store_scatter(aref, [t16], a16, mask=m)
        gk = plsc.load_gather(kref, [t16])
        ga = plsc.load_gather(aref, [t16])
        again = jnp.logical_or(
            k16 > gk, jnp.logical_and(k16 == gk, ga != a16))
        m = jnp.logical_and(again, m)


def _upd_val(kref, t16, k16):
    m = k16 > plsc.load_gather(kref, [t16])
    for _ in range(3):
        plsc.store_scatter(kref, [t16], k16, mask=m)
        m = jnp.logical_and(k16 > plsc.load_gather(kref, [t16]), m)


def _worker_id():
    return lax.axis_index("s") * _NC + lax.axis_index("c")


def _butterfly16(vecs):
    lanes = lax.iota(I32, 16)
    for mask in (8, 4, 2, 1):
        perm = jnp.bitwise_xor(lanes, mask)
        sel = (lanes & mask) == 0
        half = len(vecs) // 2
        nxt = []
        for i in range(half):
            a = vecs[i]
            b = vecs[i + half]
            ra = a + jnp.take_along_axis(a, perm, axis=0)
            rb = b + jnp.take_along_axis(b, perm, axis=0)
            nxt.append(jnp.where(sel, ra, rb))
        vecs = nxt
    return vecs[0]


def _prep_body(x1_ref, x2_ref, e1_ref, e2_ref, o1_ref, o2_ref, o3_ref, o4_ref):
    for x_ref, o_ref in ((x1_ref, o1_ref), (x2_ref, o2_ref),
                         (e1_ref, o3_ref), (e2_ref, o4_ref)):
        x = x_ref[...]
        n = jnp.sqrt(jnp.sum(x * x, axis=2))
        o_ref[...] = 1.0 / jnp.maximum(n, 1e-8)


def _inv_norms(x1p, x2p, e1p, e2p):
    g = _NP // 1024
    outs = pl.pallas_call(
        _prep_body,
        grid=(g,),
        in_specs=[
            pl.BlockSpec((8, 128, _D), lambda i: (i, 0, 0)),
            pl.BlockSpec((8, 128, _D), lambda i: (i, 0, 0)),
            pl.BlockSpec((8, 128, _DE), lambda i: (i, 0, 0)),
            pl.BlockSpec((8, 128, _DE), lambda i: (i, 0, 0)),
        ],
        out_specs=[pl.BlockSpec((8, 128), lambda i: (i, 0))] * 4,
        out_shape=[jax.ShapeDtypeStruct((8 * g, 128), F32)] * 4,
    )(x1p.reshape(8 * g, 128, _D), x2p.reshape(8 * g, 128, _D),
      e1p.reshape(8 * g, 128, _DE), e2p.reshape(8 * g, 128, _DE))
    return tuple(o.reshape(_NP) for o in outs)


def _sc_pass_a(x1, x2, srgood, engood, inv1, inv2):
    mesh = plsc.VectorSubcoreMesh(core_axis_name="c", subcore_axis_name="s")

    @functools.partial(
        pl.kernel,
        out_type=(
            jax.ShapeDtypeStruct((_NW, _NP), F32),
            jax.ShapeDtypeStruct((_NW, _NP), I32),
            jax.ShapeDtypeStruct((_NW, _NP), F32),
            jax.ShapeDtypeStruct((_NW, _NP), I32),
        ),
        mesh=mesh,
        compiler_params=pltpu.CompilerParams(needs_layout_passes=False, use_tc_tiling_on_sc=False),
        scratch_types=[
            pltpu.VMEM((_NCHMAX, _CHA), I32),
            pltpu.VMEM((_NCHMAX, _CHA), I32),
            pltpu.VMEM((_NP,), F32),
            pltpu.VMEM((_NP,), F32),
            pltpu.VMEM((_NP,), F32),
            pltpu.VMEM((_NP,), I32),
            pltpu.VMEM((_NP,), F32),
            pltpu.VMEM((_NP,), I32),
            pltpu.VMEM((_NBUF * _CHA, _D), F32),
            pltpu.VMEM((_NBUF * _CHA, _D), F32),
            pltpu.SemaphoreType.DMA,
            pltpu.SemaphoreType.DMA,
        ],
    )
    def k(x1h, x2h, srch, endh, inv1h, inv2h,
          ks1o, js1o, ks2o, is2o,
          sidx, eidx, inv1v, inv2v, bk1, ba1, bk2, ba2, x1r, x2r, sem1, sem2):
        w = _worker_id()
        cid = lax.axis_index("c")
        sid = lax.axis_index("s")
        ebase = sid * (2 * _EPW) + cid * _E0
        ecnt = jnp.where(cid == 0, _E0, _E1)
        nch = ecnt // _CHA
        chbase = ebase // _CHA
        pltpu.sync_copy(srch.at[pl.ds(chbase, _NCHMAX)], sidx)
        pltpu.sync_copy(endh.at[pl.ds(chbase, _NCHMAX)], eidx)
        pltpu.sync_copy(inv1h, inv1v)
        pltpu.sync_copy(inv2h, inv2v)

        neg = jnp.full((16,), _NEG, F32)
        zero = jnp.zeros((16,), I32)

        def init(i, _):
            bk1[pl.ds(i * 16, 16)] = neg
            bk2[pl.ds(i * 16, 16)] = neg
            ba1[pl.ds(i * 16, 16)] = zero
            ba2[pl.ds(i * 16, 16)] = zero
            return 0

        lax.fori_loop(0, _NP // 16, init, 0)

        def issue(c, b):
            pltpu.async_copy(x1h.at[sidx.at[c]],
                             x1r.at[pl.ds(b * _CHA, _CHA)], sem1)
            pltpu.async_copy(x2h.at[eidx.at[c]],
                             x2r.at[pl.ds(b * _CHA, _CHA)], sem2)

        def wait(b):
            pltpu.make_async_copy(x1h.at[sidx.at[0]],
                                  x1r.at[pl.ds(b * _CHA, _CHA)], sem1).wait()
            pltpu.make_async_copy(x2h.at[eidx.at[0]],
                                  x2r.at[pl.ds(b * _CHA, _CHA)], sem2).wait()

        def compute(c, b):
            rowoff = b * _CHA

            def grp(g, _):
                s16 = sidx[c, pl.ds(g * 16, 16)]
                e16 = eidx[c, pl.ds(g * 16, 16)]
                parts = [
                    x1r[rowoff + g * 16 + e, pl.ds(0, 16)]
                    * x2r[rowoff + g * 16 + e, pl.ds(0, 16)]
                    for e in range(16)
                ]
                for k in range(1, _D // 16):
                    for e in range(16):
                        row = rowoff + g * 16 + e
                        parts[e] = parts[e] + (x1r[row, pl.ds(k * 16, 16)]
                                               * x2r[row, pl.ds(k * 16, 16)])
                acc = _butterfly16(parts)
                i2 = plsc.load_gather(inv2v, [e16])
                i1 = plsc.load_gather(inv1v, [s16])
                _upd_pair(bk1, ba1, s16, acc * i2, e16)
                _upd_pair(bk2, ba2, e16, acc * i1, s16)
                return 0

            lax.fori_loop(0, _CHA // 16, grp, 0)

        for b in range(_NBUF):
            issue(b, b)

        def chunkn(cc, _):
            for b in range(_NBUF):
                c = cc * _NBUF + b
                wait(b)
                compute(c, b)

                @pl.when(c + _NBUF < nch)
                def _():
                    issue(c + _NBUF, b)

            return 0

        lax.fori_loop(0, nch // _NBUF, chunkn, 0)

        pltpu.sync_copy(bk1, ks1o.at[w])
        pltpu.sync_copy(ba1, js1o.at[w])
        pltpu.sync_copy(bk2, ks2o.at[w])
        pltpu.sync_copy(ba2, is2o.at[w])

    return k(x1, x2, srgood, engood, inv1, inv2)


def _sc_pass_b(ea1, ea2, srood, engood, ive1, ive2):
    mesh = plsc.VectorSubcoreMesh(core_axis_name="c", subcore_axis_name="s")

    @functools.partial(
        pl.kernel,
        out_type=(
            jax.ShapeDtypeStruct((_NW, _NP), F32),
            jax.ShapeDtypeStruct((_NW, _NP), I32),
            jax.ShapeDtypeStruct((_NW, _NP), F32),
            jax.ShapeDtypeStruct((_NW, _NP), I32),
            jax.ShapeDtypeStruct((_NW, _NP), F32),
            jax.ShapeDtypeStruct((_NW, _NP), F32),
        ),
        mesh=mesh,
        compiler_params=pltpu.CompilerParams(needs_layout_passes=False, use_tc_tiling_on_sc=False),
        scratch_types=[
            pltpu.VMEM((_NCHB, _CHB), I32),
            pltpu.VMEM((_NCHB, _CHB), I32),
            pltpu.VMEM((_NP,), F32),
            pltpu.VMEM((_NP,), F32),
            pltpu.VMEM((_NP,), F32),
            pltpu.VMEM((_NP,), I32),
            pltpu.VMEM((_NP,), F32),
            pltpu.VMEM((_NP,), I32),
            pltpu.VMEM((_NP,), F32),
            pltpu.VMEM((_NP,), F32),
            pltpu.VMEM((2 * _CHB, _DE), F32),
            pltpu.VMEM((2 * _CHB, _DE), F32),
            pltpu.SemaphoreType.DMA,
            pltpu.SemaphoreType.DMA,
        ],
    )
    def k(e1h, e2h, srch, endh, ive1h, ive2h,
          kd1o, jd1o, kd2o, id2o, kc1o, kc2o,
          sidx, eidx, ie1v, ie2v, bkd1, bad1, bkd2, bad2, bkc1, bkc2,
          e1r, e2r, sem1, sem2):
        w = _worker_id()
        pltpu.sync_copy(srch.at[pl.ds(w * _NCHB, _NCHB)], sidx)
        pltpu.sync_copy(endh.at[pl.ds(w * _NCHB, _NCHB)], eidx)
        pltpu.sync_copy(ive1h, ie1v)
        pltpu.sync_copy(ive2h, ie2v)

        neg = jnp.full((16,), _NEG, F32)
        zero = jnp.zeros((16,), I32)

        def init(i, _):
            bkd1[pl.ds(i * 16, 16)] = neg
            bkd2[pl.ds(i * 16, 16)] = neg
            bkc1[pl.ds(i * 16, 16)] = neg
            bkc2[pl.ds(i * 16, 16)] = neg
            bad1[pl.ds(i * 16, 16)] = zero
            bad2[pl.ds(i * 16, 16)] = zero
            return 0

        lax.fori_loop(0, _NP // 16, init, 0)

        def issue(c, b):
            pltpu.async_copy(e1h.at[sidx.at[c]],
                             e1r.at[pl.ds(b * _CHB, _CHB)], sem1)
            pltpu.async_copy(e2h.at[eidx.at[c]],
                             e2r.at[pl.ds(b * _CHB, _CHB)], sem2)

        def wait(b):
            pltpu.make_async_copy(e1h.at[sidx.at[0]],
                                  e1r.at[pl.ds(b * _CHB, _CHB)], sem1).wait()
            pltpu.make_async_copy(e2h.at[eidx.at[0]],
                                  e2r.at[pl.ds(b * _CHB, _CHB)], sem2).wait()

        def compute(c, b):
            rowoff = b * _CHB

            def grp(g, _):
                s16 = sidx[c, pl.ds(g * 16, 16)]
                e16 = eidx[c, pl.ds(g * 16, 16)]
                sparts = []
                tparts = []
                for e in range(16):
                    row = rowoff + g * 16 + e
                    a = e1r[row, :]
                    b_ = e2r[row, :]
                    df = a - b_ + 1e-6
                    sparts.append(df * df)
                    tparts.append(a * b_)
                sacc = _butterfly16(sparts)
                tacc = _butterfly16(tparts)
                kd = -sacc
                i1 = plsc.load_gather(ie1v, [s16])
                i2 = plsc.load_gather(ie2v, [e16])
                kc = tacc * i1 * i2
                _upd_pair(bkd1, bad1, s16, kd, e16)
                _upd_pair(bkd2, bad2, e16, kd, s16)
                _upd_val(bkc1, s16, kc)
                _upd_val(bkc2, e16, kc)
                return 0

            lax.fori_loop(0, _CHB // 16, grp, 0)

        issue(0, 0)
        issue(1, 1)

        def chunk2(cc, _):
            c0 = 2 * cc
            wait(0)
            compute(c0, 0)

            @pl.when(c0 + 2 < _NCHB)
            def _():
                issue(c0 + 2, 0)

            wait(1)
            compute(c0 + 1, 1)

            @pl.when(c0 + 3 < _NCHB)
            def _():
                issue(c0 + 3, 1)

            return 0

        lax.fori_loop(0, _NCHB // 2, chunk2, 0)

        pltpu.sync_copy(bkd1, kd1o.at[w])
        pltpu.sync_copy(bad1, jd1o.at[w])
        pltpu.sync_copy(bkd2, kd2o.at[w])
        pltpu.sync_copy(bad2, id2o.at[w])
        pltpu.sync_copy(bkc1, kc1o.at[w])
        pltpu.sync_copy(bkc2, kc2o.at[w])

    return k(ea1, ea2, srood, engood, ive1, ive2)


def _sc_pass_c(ks1, js1, ks2, is2, kd1, jd1, kd2, id2, kc1, kc2,
               x1, x2, ea1, ea2, fb):
    mesh = plsc.VectorSubcoreMesh(core_axis_name="c", subcore_axis_name="s")

    @functools.partial(
        pl.kernel,
        out_type=(
            jax.ShapeDtypeStruct((_NP, _D), F32),
            jax.ShapeDtypeStruct((_NP, _D), F32),
            jax.ShapeDtypeStruct((_NP, _DE), F32),
            jax.ShapeDtypeStruct((_NP, _DE), F32),
            jax.ShapeDtypeStruct((8, _NP), F32),
        ),
        mesh=mesh,
        compiler_params=pltpu.CompilerParams(needs_layout_passes=False, use_tc_tiling_on_sc=False),
        scratch_types=[
            pltpu.VMEM((_NW, _NPW), F32),
            pltpu.VMEM((_NW, _NPW), I32),
            pltpu.VMEM((_NPW,), F32),
            pltpu.VMEM((_NPW // 64, 64), I32),
            pltpu.VMEM((16,), I32),
            pltpu.VMEM((64, _D), F32),
            pltpu.VMEM((64, _DE), F32),
            pltpu.SemaphoreType.DMA,
        ],
    )
    def k(ks1h, js1h, ks2h, is2h, kd1h, jd1h, kd2h, id2h, kc1h, kc2h,
          x1h, x2h, e1h, e2h, fbh,
          x2go, x1go, e2go, e1go, valso,
          kslab, aslab, mk, ja, fbv, gbx, gbe, sem):
        w = _worker_id()
        nbase = w * _NPW
        pltpu.sync_copy(fbh, fbv)
        fb_all = fbv[pl.ds(0, 16)]

        def merge(khbm, ahbm, fb_lane):
            pltpu.sync_copy(khbm.at[:, pl.ds(nbase, _NPW)], kslab)
            if ahbm is not None:
                pltpu.sync_copy(ahbm.at[:, pl.ds(nbase, _NPW)], aslab)
            fbs = fb_all[fb_lane]
            for v in range(_NPW // 16):
                bk = kslab[0, pl.ds(v * 16, 16)]
                ba = aslab[0, pl.ds(v * 16, 16)]

                def red(kw, st):
                    bk_, ba_ = st
                    kk = kslab[kw, pl.ds(v * 16, 16)]
                    aa = aslab[kw, pl.ds(v * 16, 16)]
                    m = kk > bk_
                    return (jnp.where(m, kk, bk_), jnp.where(m, aa, ba_))

                bk, ba = lax.fori_loop(1, _NW, red, (bk, ba))
                emp = bk == _NEG
                ba = jnp.where(emp, jnp.full((16,), fbs, I32), ba)
                mk[pl.ds(v * 16, 16)] = bk
                ja[v // 4, pl.ds((v % 4) * 16, 16)] = ba

        def gather_rows(table, gbuf, out):
            for i in range(_NPW // 64):
                pltpu.async_copy(table.at[ja.at[i]], gbuf, sem).wait()
                pltpu.sync_copy(gbuf, out.at[pl.ds(nbase + i * 64, 64)])

        def save_vals(r):
            pltpu.sync_copy(mk, valso.at[r, pl.ds(nbase, _NPW)])

        merge(ks1h, js1h, 0)
        gather_rows(x2h, gbx, x2go)
        merge(ks2h, is2h, 1)
        gather_rows(x1h, gbx, x1go)
        merge(kd1h, jd1h, 0)
        gather_rows(e2h, gbe, e2go)
        save_vals(0)
        merge(kd2h, id2h, 1)
        gather_rows(e1h, gbe, e1go)
        save_vals(1)
        merge(kc1h, None, 0)
        save_vals(2)
        merge(kc2h, None, 1)
        save_vals(3)

    return k(ks1, js1, ks2, is2, kd1, jd1, kd2, id2, kc1, kc2,
             x1, x2, ea1, ea2, fb)


def _final_body(x1_ref, x2_ref, e1_ref, e2_ref, x2g_ref, x1g_ref,
                e2g_ref, e1g_ref, vals_ref, we_ref, be_ref, wn_ref, bn_ref,
                oe1_ref, oe2_ref, ox1_ref, ox2_ref):
    x1 = x1_ref[...]
    x2 = x2_ref[...]
    e1 = e1_ref[...]
    e2 = e2_ref[...]
    x2g = x2g_ref[...]
    x1g = x1g_ref[...]
    e2g = e2g_ref[...]
    e1g = e1g_ref[...]
    we = we_ref[...]
    wn = wn_ref[...]
    be = be_ref[0]
    bn = bn_ref[0]

    def dot(a, b):
        return lax.dot_general(a, b, (((1,), (0,)), ((), ())),
                               preferred_element_type=F32)

    oe1_ref[...] = (dot(e1, we[0:_DE]) + dot(x1, we[_DE:_DE + _D])
                    + dot(x2g, we[_DE + _D:_DE + 2 * _D]) + be)
    oe2_ref[...] = (dot(e2, we[0:_DE]) + dot(x2, we[_DE:_DE + _D])
                    + dot(x1g, we[_DE + _D:_DE + 2 * _D]) + be)

    kd1 = vals_ref[0]
    kd2 = vals_ref[1]
    kc1 = vals_ref[2]
    kc2 = vals_ref[3]
    dis1 = jnp.where(jnp.isfinite(kd1), -jnp.sqrt(jnp.maximum(-kd1, 0.0)), 0.0)
    dis2 = jnp.where(jnp.isfinite(kd2), -jnp.sqrt(jnp.maximum(-kd2, 0.0)), 0.0)
    sim1 = jnp.where(jnp.isfinite(kc1), kc1, 0.0)
    sim2 = jnp.where(jnp.isfinite(kc2), kc2, 0.0)

    ox1_ref[...] = (dot(x1, wn[0:_D]) + dot(e1, wn[_D:_D + _DE])
                    + dot(e2g, wn[_D + _DE:_D + 2 * _DE])
                    + dis1[:, None] * wn[_D + 2 * _DE]
                    + sim1[:, None] * wn[_D + 2 * _DE + 1] + bn)
    ox2_ref[...] = (dot(x2, wn[0:_D]) + dot(e2, wn[_D:_D + _DE])
                    + dot(e1g, wn[_D + _DE:_D + 2 * _DE])
                    + dis2[:, None] * wn[_D + 2 * _DE]
                    + sim2[:, None] * wn[_D + 2 * _DE + 1] + bn)


def _final(x1p, x2p, e1p, e2p, x2g, x1g, e2g, e1g, vals,
           W_edge, b_edge, W_node, b_node):
    blk = 1024
    g = _NP // blk
    kd = 168
    return pl.pallas_call(
        _final_body,
        grid=(g,),
        in_specs=[
            pl.BlockSpec((blk, _D), lambda i: (i, 0)),
            pl.BlockSpec((blk, _D), lambda i: (i, 0)),
            pl.BlockSpec((blk, _DE), lambda i: (i, 0)),
            pl.BlockSpec((blk, _DE), lambda i: (i, 0)),
            pl.BlockSpec((blk, _D), lambda i: (i, 0)),
            pl.BlockSpec((blk, _D), lambda i: (i, 0)),
            pl.BlockSpec((blk, _DE), lambda i: (i, 0)),
            pl.BlockSpec((blk, _DE), lambda i: (i, 0)),
            pl.BlockSpec((8, blk), lambda i: (0, i)),
            pl.BlockSpec((_DE + 2 * _D, _DE), lambda i: (0, 0)),
            pl.BlockSpec((1, _DE), lambda i: (0, 0)),
            pl.BlockSpec((kd, _D), lambda i: (0, 0)),
            pl.BlockSpec((1, _D), lambda i: (0, 0)),
        ],
        out_specs=[
            pl.BlockSpec((blk, _DE), lambda i: (i, 0)),
            pl.BlockSpec((blk, _DE), lambda i: (i, 0)),
            pl.BlockSpec((blk, _D), lambda i: (i, 0)),
            pl.BlockSpec((blk, _D), lambda i: (i, 0)),
        ],
        out_shape=[
            jax.ShapeDtypeStruct((_NP, _DE), F32),
            jax.ShapeDtypeStruct((_NP, _DE), F32),
            jax.ShapeDtypeStruct((_NP, _D), F32),
            jax.ShapeDtypeStruct((_NP, _D), F32),
        ],
    )(x1p, x2p, e1p, e2p, x2g, x1g, e2g, e1g, vals,
      W_edge, b_edge.reshape(1, _DE), W_node, b_node.reshape(1, _D))


def kernel(x1, x2, edge_attr1, edge_attr2, matching_idx, W_edge, b_edge,
           W_node, b_node):
    src = matching_idx[0].astype(I32)
    end = matching_idx[1].astype(I32)
    pad = _MPAD - _M
    srcp = jnp.concatenate([src, jnp.broadcast_to(src[-1], (pad,))])
    endp = jnp.concatenate([end, jnp.broadcast_to(end[-1], (pad,))])

    npad = _NP - _N
    x1p = jnp.pad(x1, ((0, npad), (0, 0)))
    x2p = jnp.pad(x2, ((0, npad), (0, 0)))
    e1p = jnp.pad(edge_attr1, ((0, npad), (0, 0)))
    e2p = jnp.pad(edge_attr2, ((0, npad), (0, 0)))

    inv1, inv2, ive1, ive2 = _inv_norms(x1p, x2p, e1p, e2p)

    ks1, js1, ks2, is2 = _sc_pass_a(
        x1, x2, srcp.reshape(-1, _CHA), endp.reshape(-1, _CHA), inv1, inv2)
    kd1, jd1, kd2, id2, kc1, kc2 = _sc_pass_b(
        edge_attr1, edge_attr2, srcp.reshape(-1, _CHB),
        endp.reshape(-1, _CHB), ive1, ive2)

    fb = jnp.concatenate([end[-1:], src[-1:], jnp.zeros((14,), I32)])
    x2g, x1g, e2g, e1g, vals = _sc_pass_c(
        ks1, js1, ks2, is2, kd1, jd1, kd2, id2, kc1, kc2,
        x1, x2, edge_attr1, edge_attr2, fb)

    Wn_pad = jnp.pad(W_node, ((0, 168 - (_D + 2 * _DE + 2)), (0, 0)))
    oe1, oe2, ox1, ox2 = _final(
        x1p, x2p, e1p, e2p, x2g, x1g, e2g, e1g, vals,
        W_edge, b_edge, Wn_pad, b_node)

    return (ox1[:_N], ox2[:_N], oe1[:_N], oe2[:_N])


# speedup vs baseline: 1.0087x; 1.0016x over previous
import functools

import jax
import jax.numpy as jnp
from jax import lax
from jax.experimental import pallas as pl
from jax.experimental.pallas import tpu as pltpu
from jax.experimental.pallas import tpu_sc as plsc

F32 = jnp.float32
I32 = jnp.int32

_N = 10000
_D = 128
_DE = 16
_M = 320000

_NC = 2
_NS = 16
_NW = _NC * _NS

_NP = 10240
_NPW = _NP // _NW
_EPW = 10240
_MP = _NW * _EPW
_MPAD = _MP + 16384
_CHA = 32
_NBUF = 4
_E0 = 12032
_E1 = 2 * 10240 - _E0
_CHB = 128
_NCHA = _EPW // _CHA
_NCHMAX = max(_E0, _E1) // _CHA
_NCHB = _EPW // _CHB

_NEG = float("-inf")


def _upd_pair(kref, aref, t16, k16, a16):
    m = k16 > plsc.load_gather(kref, [t16])
    for _ in range(3):
        plsc.store_scatter(kref, [t16], k16, mask=m)
        plsc.store_scatter(aref, [t16], a16, mask=m)
        gk = plsc.load_gather(kref, [t16])
        ga = plsc.load_gather(aref, [t16])
        again = jnp.logical_or(
            k16 > gk, jnp.logical_and(k16 == gk, ga != a16))
        m = jnp.logical_and(again, m)


def _upd_val(kref, t16, k16):
    m = k16 > plsc.load_gather(kref, [t16])
    for _ in range(3):
        plsc.store_scatter(kref, [t16], k16, mask=m)
        m = jnp.logical_and(k16 > plsc.load_gather(kref, [t16]), m)


def _worker_id():
    return lax.axis_index("s") * _NC + lax.axis_index("c")


def _butterfly16(vecs):
    lanes = lax.iota(I32, 16)
    for mask in (8, 4, 2, 1):
        perm = jnp.bitwise_xor(lanes, mask)
        sel = (lanes & mask) == 0
        half = len(vecs) // 2
        nxt = []
        for i in range(half):
            a = vecs[i]
            b = vecs[i + half]
            ra = a + jnp.take_along_axis(a, perm, axis=0)
            rb = b + jnp.take_along_axis(b, perm, axis=0)
            nxt.append(jnp.where(sel, ra, rb))
        vecs = nxt
    return vecs[0]


def _prep_body(x1_ref, x2_ref, e1_ref, e2_ref, o1_ref, o2_ref, o3_ref, o4_ref):
    for x_ref, o_ref in ((x1_ref, o1_ref), (x2_ref, o2_ref),
                         (e1_ref, o3_ref), (e2_ref, o4_ref)):
        x = x_ref[...]
        n = jnp.sqrt(jnp.sum(x * x, axis=2))
        o_ref[...] = 1.0 / jnp.maximum(n, 1e-8)


def _inv_norms(x1p, x2p, e1p, e2p):
    g = _NP // 1024
    outs = pl.pallas_call(
        _prep_body,
        grid=(g,),
        in_specs=[
            pl.BlockSpec((8, 128, _D), lambda i: (i, 0, 0)),
            pl.BlockSpec((8, 128, _D), lambda i: (i, 0, 0)),
            pl.BlockSpec((8, 128, _DE), lambda i: (i, 0, 0)),
            pl.BlockSpec((8, 128, _DE), lambda i: (i, 0, 0)),
        ],
        out_specs=[pl.BlockSpec((8, 128), lambda i: (i, 0))] * 4,
        out_shape=[jax.ShapeDtypeStruct((8 * g, 128), F32)] * 4,
    )(x1p.reshape(8 * g, 128, _D), x2p.reshape(8 * g, 128, _D),
      e1p.reshape(8 * g, 128, _DE), e2p.reshape(8 * g, 128, _DE))
    return tuple(o.reshape(_NP) for o in outs)


def _sc_pass_a(x1, x2, srgood, engood, inv1, inv2):
    mesh = plsc.VectorSubcoreMesh(core_axis_name="c", subcore_axis_name="s")

    @functools.partial(
        pl.kernel,
        out_type=(
            jax.ShapeDtypeStruct((_NW, _NP), F32),
            jax.ShapeDtypeStruct((_NW, _NP), I32),
            jax.ShapeDtypeStruct((_NW, _NP), F32),
            jax.ShapeDtypeStruct((_NW, _NP), I32),
        ),
        mesh=mesh,
        compiler_params=pltpu.CompilerParams(needs_layout_passes=False, use_tc_tiling_on_sc=False),
        scratch_types=[
            pltpu.VMEM((_NCHMAX, _CHA), I32),
            pltpu.VMEM((_NCHMAX, _CHA), I32),
            pltpu.VMEM((_NP,), F32),
            pltpu.VMEM((_NP,), F32),
            pltpu.VMEM((_NP,), F32),
            pltpu.VMEM((_NP,), I32),
            pltpu.VMEM((_NP,), F32),
            pltpu.VMEM((_NP,), I32),
            pltpu.VMEM((_NBUF * _CHA, _D), F32),
            pltpu.VMEM((_NBUF * _CHA, _D), F32),
            pltpu.SemaphoreType.DMA,
            pltpu.SemaphoreType.DMA,
        ],
    )
    def k(x1h, x2h, srch, endh, inv1h, inv2h,
          ks1o, js1o, ks2o, is2o,
          sidx, eidx, inv1v, inv2v, bk1, ba1, bk2, ba2, x1r, x2r, sem1, sem2):
        w = _worker_id()
        cid = lax.axis_index("c")
        sid = lax.axis_index("s")
        ebase = sid * (2 * _EPW) + cid * _E0
        ecnt = jnp.where(cid == 0, _E0, _E1)
        nch = ecnt // _CHA
        chbase = ebase // _CHA
        pltpu.sync_copy(srch.at[pl.ds(chbase, _NCHMAX)], sidx)
        pltpu.sync_copy(endh.at[pl.ds(chbase, _NCHMAX)], eidx)
        pltpu.sync_copy(inv1h, inv1v)
        pltpu.sync_copy(inv2h, inv2v)

        neg = jnp.full((16,), _NEG, F32)
        zero = jnp.zeros((16,), I32)

        def init(i, _):
            bk1[pl.ds(i * 16, 16)] = neg
            bk2[pl.ds(i * 16, 16)] = neg
            ba1[pl.ds(i * 16, 16)] = zero
            ba2[pl.ds(i * 16, 16)] = zero
            return 0

        lax.fori_loop(0, _NP // 16, init, 0)

        def issue(c, b):
            pltpu.async_copy(x1h.at[sidx.at[c]],
                             x1r.at[pl.ds(b * _CHA, _CHA)], sem1)
            pltpu.async_copy(x2h.at[eidx.at[c]],
                             x2r.at[pl.ds(b * _CHA, _CHA)], sem2)

        def wait(b):
            pltpu.make_async_copy(x1h.at[sidx.at[0]],
                                  x1r.at[pl.ds(b * _CHA, _CHA)], sem1).wait()
            pltpu.make_async_copy(x2h.at[eidx.at[0]],
                                  x2r.at[pl.ds(b * _CHA, _CHA)], sem2).wait()

        def compute(c, b):
            rowoff = b * _CHA

            def grp(g, _):
                s16 = sidx[c, pl.ds(g * 16, 16)]
                e16 = eidx[c, pl.ds(g * 16, 16)]
                parts = [
                    x1r[rowoff + g * 16 + e, pl.ds(0, 16)]
                    * x2r[rowoff + g * 16 + e, pl.ds(0, 16)]
                    for e in range(16)
                ]
                for k in range(1, _D // 16):
                    for e in range(16):
                        row = rowoff + g * 16 + e
                        parts[e] = parts[e] + (x1r[row, pl.ds(k * 16, 16)]
                                               * x2r[row, pl.ds(k * 16, 16)])
                acc = _butterfly16(parts)
                i2 = plsc.load_gather(inv2v, [e16])
                i1 = plsc.load_gather(inv1v, [s16])
                _upd_pair(bk1, ba1, s16, acc * i2, e16)
                _upd_pair(bk2, ba2, e16, acc * i1, s16)
                return 0

            lax.fori_loop(0, _CHA // 16, grp, 0)

        for b in range(_NBUF):
            issue(b, b)

        def chunkn(cc, _):
            for b in range(_NBUF):
                c = cc * _NBUF + b
                wait(b)
                compute(c, b)

                @pl.when(c + _NBUF < nch)
                def _():
                    issue(c + _NBUF, b)

            return 0

        lax.fori_loop(0, nch // _NBUF, chunkn, 0)

        pltpu.sync_copy(bk1, ks1o.at[w])
        pltpu.sync_copy(ba1, js1o.at[w])
        pltpu.sync_copy(bk2, ks2o.at[w])
        pltpu.sync_copy(ba2, is2o.at[w])

    return k(x1, x2, srgood, engood, inv1, inv2)


def _sc_pass_b(ea1, ea2, srood, engood, ive1, ive2):
    mesh = plsc.VectorSubcoreMesh(core_axis_name="c", subcore_axis_name="s")

    @functools.partial(
        pl.kernel,
        out_type=(
            jax.ShapeDtypeStruct((_NW, _NP), F32),
            jax.ShapeDtypeStruct((_NW, _NP), I32),
            jax.ShapeDtypeStruct((_NW, _NP), F32),
            jax.ShapeDtypeStruct((_NW, _NP), I32),
            jax.ShapeDtypeStruct((_NW, _NP), F32),
            jax.ShapeDtypeStruct((_NW, _NP), F32),
        ),
        mesh=mesh,
        compiler_params=pltpu.CompilerParams(needs_layout_passes=False, use_tc_tiling_on_sc=False),
        scratch_types=[
            pltpu.VMEM((_NCHB, _CHB), I32),
            pltpu.VMEM((_NCHB, _CHB), I32),
            pltpu.VMEM((_NP,), F32),
            pltpu.VMEM((_NP,), F32),
            pltpu.VMEM((_NP,), F32),
            pltpu.VMEM((_NP,), I32),
            pltpu.VMEM((_NP,), F32),
            pltpu.VMEM((_NP,), I32),
            pltpu.VMEM((_NP,), F32),
            pltpu.VMEM((_NP,), F32),
            pltpu.VMEM((2 * _CHB, _DE), F32),
            pltpu.VMEM((2 * _CHB, _DE), F32),
            pltpu.SemaphoreType.DMA,
            pltpu.SemaphoreType.DMA,
        ],
    )
    def k(e1h, e2h, srch, endh, ive1h, ive2h,
          kd1o, jd1o, kd2o, id2o, kc1o, kc2o,
          sidx, eidx, ie1v, ie2v, bkd1, bad1, bkd2, bad2, bkc1, bkc2,
          e1r, e2r, sem1, sem2):
        w = _worker_id()
        pltpu.sync_copy(srch.at[pl.ds(w * _NCHB, _NCHB)], sidx)
        pltpu.sync_copy(endh.at[pl.ds(w * _NCHB, _NCHB)], eidx)
        pltpu.sync_copy(ive1h, ie1v)
        pltpu.sync_copy(ive2h, ie2v)

        neg = jnp.full((16,), _NEG, F32)
        zero = jnp.zeros((16,), I32)

        def init(i, _):
            bkd1[pl.ds(i * 16, 16)] = neg
            bkd2[pl.ds(i * 16, 16)] = neg
            bkc1[pl.ds(i * 16, 16)] = neg
            bkc2[pl.ds(i * 16, 16)] = neg
            bad1[pl.ds(i * 16, 16)] = zero
            bad2[pl.ds(i * 16, 16)] = zero
            return 0

        lax.fori_loop(0, _NP // 16, init, 0)

        def issue(c, b):
            pltpu.async_copy(e1h.at[sidx.at[c]],
                             e1r.at[pl.ds(b * _CHB, _CHB)], sem1)
            pltpu.async_copy(e2h.at[eidx.at[c]],
                             e2r.at[pl.ds(b * _CHB, _CHB)], sem2)

        def wait(b):
            pltpu.make_async_copy(e1h.at[sidx.at[0]],
                                  e1r.at[pl.ds(b * _CHB, _CHB)], sem1).wait()
            pltpu.make_async_copy(e2h.at[eidx.at[0]],
                                  e2r.at[pl.ds(b * _CHB, _CHB)], sem2).wait()

        def compute(c, b):
            rowoff = b * _CHB

            def grp(g, _):
                s16 = sidx[c, pl.ds(g * 16, 16)]
                e16 = eidx[c, pl.ds(g * 16, 16)]
                sparts = []
                tparts = []
                for e in range(16):
                    row = rowoff + g * 16 + e
                    a = e1r[row, :]
                    b_ = e2r[row, :]
                    df = a - b_ + 1e-6
                    sparts.append(df * df)
                    tparts.append(a * b_)
                sacc = _butterfly16(sparts)
                tacc = _butterfly16(tparts)
                kd = -sacc
                i1 = plsc.load_gather(ie1v, [s16])
                i2 = plsc.load_gather(ie2v, [e16])
                kc = tacc * i1 * i2
                _upd_pair(bkd1, bad1, s16, kd, e16)
                _upd_pair(bkd2, bad2, e16, kd, s16)
                _upd_val(bkc1, s16, kc)
                _upd_val(bkc2, e16, kc)
                return 0

            lax.fori_loop(0, _CHB // 16, grp, 0)

        issue(0, 0)
        issue(1, 1)

        def chunk2(cc, _):
            c0 = 2 * cc
            wait(0)
            compute(c0, 0)

            @pl.when(c0 + 2 < _NCHB)
            def _():
                issue(c0 + 2, 0)

            wait(1)
            compute(c0 + 1, 1)

            @pl.when(c0 + 3 < _NCHB)
            def _():
                issue(c0 + 3, 1)

            return 0

        lax.fori_loop(0, _NCHB // 2, chunk2, 0)

        pltpu.sync_copy(bkd1, kd1o.at[w])
        pltpu.sync_copy(bad1, jd1o.at[w])
        pltpu.sync_copy(bkd2, kd2o.at[w])
        pltpu.sync_copy(bad2, id2o.at[w])
        pltpu.sync_copy(bkc1, kc1o.at[w])
        pltpu.sync_copy(bkc2, kc2o.at[w])

    return k(ea1, ea2, srood, engood, ive1, ive2)


def _sc_pass_c(ks1, js1, ks2, is2, kd1, jd1, kd2, id2, kc1, kc2,
               x1, x2, ea1, ea2, fb):
    mesh = plsc.VectorSubcoreMesh(core_axis_name="c", subcore_axis_name="s")

    @functools.partial(
        pl.kernel,
        out_type=(
            jax.ShapeDtypeStruct((_NP, _D), F32),
            jax.ShapeDtypeStruct((_NP, _D), F32),
            jax.ShapeDtypeStruct((_NP, _DE), F32),
            jax.ShapeDtypeStruct((_NP, _DE), F32),
            jax.ShapeDtypeStruct((8, _NP), F32),
        ),
        mesh=mesh,
        compiler_params=pltpu.CompilerParams(needs_layout_passes=False, use_tc_tiling_on_sc=False),
        scratch_types=[
            pltpu.VMEM((_NW, _NPW), F32),
            pltpu.VMEM((_NW, _NPW), I32),
            pltpu.VMEM((_NPW,), F32),
            pltpu.VMEM((_NPW // 64, 64), I32),
            pltpu.VMEM((16,), I32),
            pltpu.VMEM((64, _D), F32),
            pltpu.VMEM((64, _DE), F32),
            pltpu.SemaphoreType.DMA,
        ],
    )
    def k(ks1h, js1h, ks2h, is2h, kd1h, jd1h, kd2h, id2h, kc1h, kc2h,
          x1h, x2h, e1h, e2h, fbh,
          x2go, x1go, e2go, e1go, valso,
          kslab, aslab, mk, ja, fbv, gbx, gbe, sem):
        w = _worker_id()
        nbase = w * _NPW
        pltpu.sync_copy(fbh, fbv)
        fb_all = fbv[pl.ds(0, 16)]

        def merge(khbm, ahbm, fb_lane):
            pltpu.sync_copy(khbm.at[:, pl.ds(nbase, _NPW)], kslab)
            if ahbm is not None:
                pltpu.sync_copy(ahbm.at[:, pl.ds(nbase, _NPW)], aslab)
            fbs = fb_all[fb_lane]
            for v in range(_NPW // 16):
                bk = kslab[0, pl.ds(v * 16, 16)]
                ba = aslab[0, pl.ds(v * 16, 16)]

                def red(kw, st):
                    bk_, ba_ = st
                    kk = kslab[kw, pl.ds(v * 16, 16)]
                    aa = aslab[kw, pl.ds(v * 16, 16)]
                    m = kk > bk_
                    return (jnp.where(m, kk, bk_), jnp.where(m, aa, ba_))

                bk, ba = lax.fori_loop(1, _NW, red, (bk, ba))
                emp = bk == _NEG
                ba = jnp.where(emp, jnp.full((16,), fbs, I32), ba)
                mk[pl.ds(v * 16, 16)] = bk
                ja[v // 4, pl.ds((v % 4) * 16, 16)] = ba

        def gather_rows(table, gbuf, out):
            for i in range(_NPW // 64):
                pltpu.async_copy(table.at[ja.at[i]], gbuf, sem).wait()
                pltpu.sync_copy(gbuf, out.at[pl.ds(nbase + i * 64, 64)])

        def save_vals(r):
            pltpu.sync_copy(mk, valso.at[r, pl.ds(nbase, _NPW)])

        merge(ks1h, js1h, 0)
        gather_rows(x2h, gbx, x2go)
        merge(ks2h, is2h, 1)
        gather_rows(x1h, gbx, x1go)
        merge(kd1h, jd1h, 0)
        gather_rows(e2h, gbe, e2go)
        save_vals(0)
        merge(kd2h, id2h, 1)
        gather_rows(e1h, gbe, e1go)
        save_vals(1)
        merge(kc1h, None, 0)
        save_vals(2)
        merge(kc2h, None, 1)
        save_vals(3)

    return k(ks1, js1, ks2, is2, kd1, jd1, kd2, id2, kc1, kc2,
             x1, x2, ea1, ea2, fb)


def _final_body(x1_ref, x2_ref, e1_ref, e2_ref, x2g_ref, x1g_ref,
                e2g_ref, e1g_ref, vals_ref, we_ref, be_ref, wn_ref, bn_ref,
                oe1_ref, oe2_ref, ox1_ref, ox2_ref):
    x1 = x1_ref[...]
    x2 = x2_ref[...]
    e1 = e1_ref[...]
    e2 = e2_ref[...]
    x2g = x2g_ref[...]
    x1g = x1g_ref[...]
    e2g = e2g_ref[...]
    e1g = e1g_ref[...]
    we = we_ref[...]
    wn = wn_ref[...]
    be = be_ref[0]
    bn = bn_ref[0]

    def dot(a, b):
        return lax.dot_general(a, b, (((1,), (0,)), ((), ())),
                               preferred_element_type=F32)

    oe1_ref[...] = (dot(e1, we[0:_DE]) + dot(x1, we[_DE:_DE + _D])
                    + dot(x2g, we[_DE + _D:_DE + 2 * _D]) + be)
    oe2_ref[...] = (dot(e2, we[0:_DE]) + dot(x2, we[_DE:_DE + _D])
                    + dot(x1g, we[_DE + _D:_DE + 2 * _D]) + be)

    kd1 = vals_ref[0]
    kd2 = vals_ref[1]
    kc1 = vals_ref[2]
    kc2 = vals_ref[3]
    dis1 = jnp.where(jnp.isfinite(kd1), -jnp.sqrt(jnp.maximum(-kd1, 0.0)), 0.0)
    dis2 = jnp.where(jnp.isfinite(kd2), -jnp.sqrt(jnp.maximum(-kd2, 0.0)), 0.0)
    sim1 = jnp.where(jnp.isfinite(kc1), kc1, 0.0)
    sim2 = jnp.where(jnp.isfinite(kc2), kc2, 0.0)

    ox1_ref[...] = (dot(x1, wn[0:_D]) + dot(e1, wn[_D:_D + _DE])
                    + dot(e2g, wn[_D + _DE:_D + 2 * _DE])
                    + dis1[:, None] * wn[_D + 2 * _DE]
                    + sim1[:, None] * wn[_D + 2 * _DE + 1] + bn)
    ox2_ref[...] = (dot(x2, wn[0:_D]) + dot(e2, wn[_D:_D + _DE])
                    + dot(e1g, wn[_D + _DE:_D + 2 * _DE])
                    + dis2[:, None] * wn[_D + 2 * _DE]
                    + sim2[:, None] * wn[_D + 2 * _DE + 1] + bn)


def _final(x1p, x2p, e1p, e2p, x2g, x1g, e2g, e1g, vals,
           W_edge, b_edge, W_node, b_node):
    blk = 1024
    g = _NP // blk
    kd = 168
    return pl.pallas_call(
        _final_body,
        grid=(g,),
        in_specs=[
            pl.BlockSpec((blk, _D), lambda i: (i, 0)),
            pl.BlockSpec((blk, _D), lambda i: (i, 0)),
            pl.BlockSpec((blk, _DE), lambda i: (i, 0)),
            pl.BlockSpec((blk, _DE), lambda i: (i, 0)),
            pl.BlockSpec((blk, _D), lambda i: (i, 0)),
            pl.BlockSpec((blk, _D), lambda i: (i, 0)),
            pl.BlockSpec((blk, _DE), lambda i: (i, 0)),
            pl.BlockSpec((blk, _DE), lambda i: (i, 0)),
            pl.BlockSpec((8, blk), lambda i: (0, i)),
            pl.BlockSpec((_DE + 2 * _D, _DE), lambda i: (0, 0)),
            pl.BlockSpec((1, _DE), lambda i: (0, 0)),
            pl.BlockSpec((kd, _D), lambda i: (0, 0)),
            pl.BlockSpec((1, _D), lambda i: (0, 0)),
        ],
        out_specs=[
            pl.BlockSpec((blk, _DE), lambda i: (i, 0)),
            pl.BlockSpec((blk, _DE), lambda i: (i, 0)),
            pl.BlockSpec((blk, _D), lambda i: (i, 0)),
            pl.BlockSpec((blk, _D), lambda i: (i, 0)),
        ],
        out_shape=[
            jax.ShapeDtypeStruct((_NP, _DE), F32),
            jax.ShapeDtypeStruct((_NP, _DE), F32),
            jax.ShapeDtypeStruct((_NP, _D), F32),
            jax.ShapeDtypeStruct((_NP, _D), F32),
        ],
    )(x1p, x2p, e1p, e2p, x2g, x1g, e2g, e1g, vals,
      W_edge, b_edge.reshape(1, _DE), W_node, b_node.reshape(1, _D))


def kernel(x1, x2, edge_attr1, edge_attr2, matching_idx, W_edge, b_edge,
           W_node, b_node):
    src = matching_idx[0].astype(I32)
    end = matching_idx[1].astype(I32)
    pad = _MPAD - _M
    srcp = jnp.concatenate([src, jnp.broadcast_to(src[-1], (pad,))])
    endp = jnp.concatenate([end, jnp.broadcast_to(end[-1], (pad,))])

    npad = _NP - _N
    x1p = jnp.pad(x1, ((0, npad), (0, 0)))
    x2p = jnp.pad(x2, ((0, npad), (0, 0)))
    e1p = jnp.pad(edge_attr1, ((0, npad), (0, 0)))
    e2p = jnp.pad(edge_attr2, ((0, npad), (0, 0)))

    inv1, inv2, ive1, ive2 = _inv_norms(x1p, x2p, e1p, e2p)

    ks1, js1, ks2, is2 = _sc_pass_a(
        x1, x2, srcp.reshape(-1, _CHA), endp.reshape(-1, _CHA), inv1, inv2)
    kd1, jd1, kd2, id2, kc1, kc2 = _sc_pass_b(
        edge_attr1, edge_attr2, srcp.reshape(-1, _CHB),
        endp.reshape(-1, _CHB), ive1, ive2)

    fb = jnp.concatenate([end[-1:], src[-1:], jnp.zeros((14,), I32)])
    x2g, x1g, e2g, e1g, vals = _sc_pass_c(
        ks1, js1, ks2, is2, kd1, jd1, kd2, id2, kc1, kc2,
        x1, x2, edge_attr1, edge_attr2, fb)

    Wn_pad = jnp.pad(W_node, ((0, 168 - (_D + 2 * _DE + 2)), (0, 0)))
    oe1, oe2, ox1, ox2 = _final(
        x1p, x2p, e1p, e2p, x2g, x1g, e2g, e1g, vals,
        W_edge, b_edge, Wn_pad, b_node)

    return (ox1[:_N], ox2[:_N], oe1[:_N], oe2[:_N])
